# Initial kernel scaffold; baseline (speedup 1.0000x reference)
import jax
import jax.numpy as jnp
from jax.experimental import pallas as pl


def kernel(x, edge_index, W1, b1, W2, b2):
    raise NotImplementedError("write your pallas kernel here")


# speedup vs baseline: 7.2131x; 7.2131x over previous
import functools

import jax
import jax.numpy as jnp
from jax import lax
from jax.experimental import pallas as pl
from jax.experimental.pallas import tpu as pltpu
from jax.experimental.pallas import tpu_sc as plsc

N = 10000
E = 320000
D = 128

NC = 2
NS = 16
NW = NC * NS

C = 128
CHUNKS = E // C
FULL = CHUNKS // NW
REM = CHUNKS % NW

ACC_ROWS = 10240
ROWS_PER_TILE = ACC_ROWS // NS
DUMMY = N


def _sc_body(row_hbm, col_hbm, x_hbm, out_hbm, row_v, col_v, rows_v, acc, sem):
    c = lax.axis_index("c")
    s = lax.axis_index("s")
    wid = c * NS + s

    def _zero_row(r, carry):
        for j in range(D // 16):
            rows_v[r, pl.ds(j * 16, 16)] = jnp.zeros((16,), jnp.float32)
        return carry

    lax.fori_loop(0, C, _zero_row, 0, unroll=False)
    for b in range(ROWS_PER_TILE // C):
        pltpu.sync_copy(rows_v, acc.at[pl.ds(s * ROWS_PER_TILE + b * C, C)])
    plsc.subcore_barrier()

    def _chunk(k, wid):
        base = k * C
        pltpu.sync_copy(row_hbm.at[pl.ds(base, C)], row_v)
        pltpu.sync_copy(col_hbm.at[pl.ds(base, C)], col_v)
        for j in range(C // 16):
            rv = row_v[pl.ds(j * 16, 16)]
            cv = col_v[pl.ds(j * 16, 16)]
            row_v[pl.ds(j * 16, 16)] = jnp.where(rv == cv, DUMMY, rv)
        pltpu.async_copy(x_hbm.at[col_v], rows_v, sem).wait()
        pltpu.sync_copy(rows_v, acc.at[row_v], add=True)

    def _loop(k, wid):
        _chunk(k * NW + wid, wid)
        return wid

    lax.fori_loop(0, FULL, _loop, wid, unroll=False)

    @pl.when(wid < REM)
    def _tail():
        _chunk(FULL * NW + wid, wid)

    plsc.subcore_barrier()

    for b in range(ROWS_PER_TILE // C):
        off = s * ROWS_PER_TILE + b * C
        pltpu.sync_copy(acc.at[pl.ds(off, C)], out_hbm.at[c, pl.ds(off, C)])


_sc_aggregate = functools.partial(
    pl.kernel,
    mesh=plsc.VectorSubcoreMesh(core_axis_name="c", subcore_axis_name="s"),
    out_type=jax.ShapeDtypeStruct((NC, ACC_ROWS, D), jnp.float32),
    scratch_types=[
        pltpu.VMEM((C,), jnp.int32),
        pltpu.VMEM((C,), jnp.int32),
        pltpu.VMEM((C, D), jnp.float32),
        pltpu.VMEM_SHARED((ACC_ROWS, D), jnp.float32),
        pltpu.SemaphoreType.DMA,
    ],
)(_sc_body)


def _mlp_body(x_ref, p_ref, w1_ref, b1_ref, w2_ref, b2_ref, o_ref):
    out = x_ref[...] + p_ref[0] + p_ref[1]
    h = jnp.dot(out, w1_ref[...], preferred_element_type=jnp.float32)
    h = jnp.maximum(h + b1_ref[...], 0.0)
    y = jnp.dot(h, w2_ref[...], preferred_element_type=jnp.float32)
    o_ref[...] = y + b2_ref[...]


MB = 2000


def _mlp(x, partials, W1, b1, W2, b2):
    grid = (N // MB,)
    return pl.pallas_call(
        _mlp_body,
        grid=grid,
        in_specs=[
            pl.BlockSpec((MB, D), lambda i: (i, 0)),
            pl.BlockSpec((NC, MB, D), lambda i: (0, i, 0)),
            pl.BlockSpec((D, D), lambda i: (0, 0)),
            pl.BlockSpec((1, D), lambda i: (0, 0)),
            pl.BlockSpec((D, D), lambda i: (0, 0)),
            pl.BlockSpec((1, D), lambda i: (0, 0)),
        ],
        out_specs=pl.BlockSpec((MB, D), lambda i: (i, 0)),
        out_shape=jax.ShapeDtypeStruct((N, D), jnp.float32),
    )(x, partials, W1, b1.reshape(1, D), W2, b2.reshape(1, D))


def kernel(x, edge_index, W1, b1, W2, b2):
    row = edge_index[0].astype(jnp.int32)
    col = edge_index[1].astype(jnp.int32)
    partials = _sc_aggregate(row, col, x)
    return _mlp(x, partials, W1, b1, W2, b2)
